# baseline (speedup 1.0000x reference)
import functools

import jax
import jax.numpy as jnp
from jax import lax
from jax.experimental import pallas as pl
from jax.experimental.pallas import tpu as pltpu
from jax.experimental.pallas import tpu_sc as plsc

B = 32
C = 256
P = 4096
S = 64
L = 16
MINPIX = 16.0
CHUNK_C = 8
NCHUNK = C // CHUNK_C


def _sc_body(seg_hbm, fm_hbm, out_hbm, cnt_hbm,
             ids_v, idsx_v, acc_v, cntf_v, cnti_v, scale_v, bufa, bufb,
             sema, semb):
    nc = 2
    wid = lax.axis_index("s") * nc + lax.axis_index("c")
    b = wid

    pltpu.sync_copy(seg_hbm.at[b], ids_v)

    zeros = jnp.zeros((L,), jnp.float32)
    ones = jnp.ones((L,), jnp.float32)

    def zbody(i, _):
        acc_v[pl.ds(i * L, L)] = zeros
        return 0
    lax.fori_loop(0, (S * C) // L, zbody, 0)

    for i in range(S // L):
        cntf_v[pl.ds(i * L, L)] = zeros

    def cbody(g, _):
        ids = ids_v[pl.ds(g * L, L)]
        plsc.addupdate_scatter(cntf_v, [ids], ones)
        idsx_v[pl.ds(g * L, L)] = ids * C
        return 0
    lax.fori_loop(0, P // L, cbody, 0)

    cpy_a = pltpu.make_async_copy(
        fm_hbm.at[b, pl.ds(0, CHUNK_C * P)], bufa, sema)

    def start(chunk, buf, sem):
        pltpu.make_async_copy(
            fm_hbm.at[b, pl.ds(chunk * (CHUNK_C * P), CHUNK_C * P)],
            buf, sem).start()

    start(0, bufa, sema)

    def compute(chunk, buf):
        c0 = chunk * CHUNK_C
        c0v = jnp.full((L,), c0, jnp.int32)

        def gbody(g, _):
            base = idsx_v[pl.ds(g * L, L)] + c0v
            off = g * L
            for cc in range(CHUNK_C):
                vals = buf[pl.ds(off + cc * P, L)]
                plsc.addupdate_scatter(acc_v, [base + cc], vals)
            return 0
        lax.fori_loop(0, P // L, gbody, 0)

    def mbody(k, _):
        start(2 * k + 1, bufb, semb)
        pltpu.make_async_copy(
            fm_hbm.at[b, pl.ds(0, CHUNK_C * P)], bufa, sema).wait()
        compute(2 * k, bufa)

        @pl.when(k < NCHUNK // 2 - 1)
        def _():
            start(2 * k + 2, bufa, sema)

        pltpu.make_async_copy(
            fm_hbm.at[b, pl.ds(0, CHUNK_C * P)], bufb, semb).wait()
        compute(2 * k + 1, bufb)
        return 0

    lax.fori_loop(0, NCHUNK // 2, mbody, 0)

    for i in range(S // L):
        cnt = cntf_v[pl.ds(i * L, L)]
        sc = jnp.where(cnt >= MINPIX, 1.0 / jnp.maximum(cnt, 1.0), 0.0)
        scale_v[pl.ds(i * L, L)] = sc
        cnti_v[pl.ds(i * L, L)] = cnt.astype(jnp.int32)

    def sbody(i, _):
        s = i // (C // L)
        sv = plsc.load_gather(scale_v, [jnp.full((L,), s, jnp.int32)])
        j = i * L
        acc_v[pl.ds(j, L)] = acc_v[pl.ds(j, L)] * sv
        return 0
    lax.fori_loop(0, (S * C) // L, sbody, 0)

    pltpu.sync_copy(acc_v, out_hbm.at[b])
    pltpu.sync_copy(cnti_v, cnt_hbm.at[b])


@jax.jit
def _sc_call(seg, fm):
    mesh = plsc.VectorSubcoreMesh(core_axis_name="c", subcore_axis_name="s")
    f = functools.partial(
        pl.kernel,
        mesh=mesh,
        compiler_params=pltpu.CompilerParams(needs_layout_passes=False),
        out_type=[
            jax.ShapeDtypeStruct((B, S * C), jnp.float32),
            jax.ShapeDtypeStruct((B, S), jnp.int32),
        ],
        scratch_types=[
            pltpu.VMEM((P,), jnp.int32),
            pltpu.VMEM((P,), jnp.int32),
            pltpu.VMEM((S * C,), jnp.float32),
            pltpu.VMEM((S,), jnp.float32),
            pltpu.VMEM((S,), jnp.int32),
            pltpu.VMEM((S,), jnp.float32),
            pltpu.VMEM((CHUNK_C * P,), jnp.float32),
            pltpu.VMEM((CHUNK_C * P,), jnp.float32),
            pltpu.SemaphoreType.DMA,
            pltpu.SemaphoreType.DMA,
        ],
    )(_sc_body)
    return f(seg, fm)


def kernel(segment_ids, sam_encoder_embeddings):
    fm = jnp.squeeze(sam_encoder_embeddings, axis=1).reshape(B, C * P)
    seg = segment_ids.reshape(B, P)
    out, cnt = _sc_call(seg, fm)
    valid = cnt >= int(MINPIX)
    return out.reshape(B, S, C), valid


# speedup vs baseline: 1.1844x; 1.1844x over previous
import functools

import jax
import jax.numpy as jnp
from jax import lax
from jax.experimental import pallas as pl
from jax.experimental.pallas import tpu as pltpu
from jax.experimental.pallas import tpu_sc as plsc

B = 32
C = 256
P = 4096
S = 64
L = 16
MINPIX = 16.0
CHUNK_C = 8
NCHUNK = C // CHUNK_C


def _sc_body(seg_hbm, fm_hbm, out_hbm, cnt_hbm,
             ids_v, idsx_v, acc_v, cntf_v, cnti_v, scale_v, bufa, bufb,
             sema, semb):
    nc = 2
    wid = lax.axis_index("s") * nc + lax.axis_index("c")
    b = wid

    pltpu.sync_copy(seg_hbm.at[b], ids_v)

    zeros = jnp.zeros((L,), jnp.float32)
    ones = jnp.ones((L,), jnp.float32)

    @plsc.parallel_loop(0, (S * C) // L, unroll=8)
    def _(i):
        acc_v[pl.ds(i * L, L)] = zeros

    for i in range(S // L):
        cntf_v[pl.ds(i * L, L)] = zeros

    @plsc.parallel_loop(0, P // L, unroll=4)
    def _(g):
        ids = ids_v[pl.ds(g * L, L)]
        plsc.addupdate_scatter(cntf_v, [ids], ones)
        idsx_v[pl.ds(g * L, L)] = ids * C

    cpy_a = pltpu.make_async_copy(
        fm_hbm.at[b, pl.ds(0, CHUNK_C * P)], bufa, sema)

    def start(chunk, buf, sem):
        pltpu.make_async_copy(
            fm_hbm.at[b, pl.ds(chunk * (CHUNK_C * P), CHUNK_C * P)],
            buf, sem).start()

    start(0, bufa, sema)

    def compute(chunk, buf):
        c0 = chunk * CHUNK_C
        c0v = jnp.full((L,), c0, jnp.int32)

        @plsc.parallel_loop(0, P // L, unroll=4)
        def _(g):
            base = idsx_v[pl.ds(g * L, L)] + c0v
            off = g * L
            for cc in range(CHUNK_C):
                vals = buf[pl.ds(off + cc * P, L)]
                plsc.addupdate_scatter(acc_v, [base + cc], vals)

    def mbody(k, _):
        start(2 * k + 1, bufb, semb)
        pltpu.make_async_copy(
            fm_hbm.at[b, pl.ds(0, CHUNK_C * P)], bufa, sema).wait()
        compute(2 * k, bufa)

        @pl.when(k < NCHUNK // 2 - 1)
        def _():
            start(2 * k + 2, bufa, sema)

        pltpu.make_async_copy(
            fm_hbm.at[b, pl.ds(0, CHUNK_C * P)], bufb, semb).wait()
        compute(2 * k + 1, bufb)
        return 0

    lax.fori_loop(0, NCHUNK // 2, mbody, 0)

    for i in range(S // L):
        cnt = cntf_v[pl.ds(i * L, L)]
        sc = jnp.where(cnt >= MINPIX, 1.0 / jnp.maximum(cnt, 1.0), 0.0)
        scale_v[pl.ds(i * L, L)] = sc
        cnti_v[pl.ds(i * L, L)] = cnt.astype(jnp.int32)

    @plsc.parallel_loop(0, (S * C) // L, unroll=4)
    def _(i):
        s = i // (C // L)
        sv = plsc.load_gather(scale_v, [jnp.full((L,), s, jnp.int32)])
        j = i * L
        acc_v[pl.ds(j, L)] = acc_v[pl.ds(j, L)] * sv

    pltpu.sync_copy(acc_v, out_hbm.at[b])
    pltpu.sync_copy(cnti_v, cnt_hbm.at[b])


@jax.jit
def _sc_call(seg, fm):
    mesh = plsc.VectorSubcoreMesh(core_axis_name="c", subcore_axis_name="s")
    f = functools.partial(
        pl.kernel,
        mesh=mesh,
        compiler_params=pltpu.CompilerParams(needs_layout_passes=False),
        out_type=[
            jax.ShapeDtypeStruct((B, S * C), jnp.float32),
            jax.ShapeDtypeStruct((B, S), jnp.int32),
        ],
        scratch_types=[
            pltpu.VMEM((P,), jnp.int32),
            pltpu.VMEM((P,), jnp.int32),
            pltpu.VMEM((S * C,), jnp.float32),
            pltpu.VMEM((S,), jnp.float32),
            pltpu.VMEM((S,), jnp.int32),
            pltpu.VMEM((S,), jnp.float32),
            pltpu.VMEM((CHUNK_C * P,), jnp.float32),
            pltpu.VMEM((CHUNK_C * P,), jnp.float32),
            pltpu.SemaphoreType.DMA,
            pltpu.SemaphoreType.DMA,
        ],
    )(_sc_body)
    return f(seg, fm)


def kernel(segment_ids, sam_encoder_embeddings):
    fm = jnp.squeeze(sam_encoder_embeddings, axis=1).reshape(B, C * P)
    seg = segment_ids.reshape(B, P)
    out, cnt = _sc_call(seg, fm)
    valid = cnt >= int(MINPIX)
    return out.reshape(B, S, C), valid


# speedup vs baseline: 2.4290x; 2.0509x over previous
import functools

import jax
import jax.numpy as jnp
from jax import lax
from jax.experimental import pallas as pl
from jax.experimental.pallas import tpu as pltpu
from jax.experimental.pallas import tpu_sc as plsc

B = 32
C = 256
P = 4096
S = 64
L = 16
MINPIX = 16.0
CHUNK_C = 8
NCHUNK = C // CHUNK_C


def _sc_body(seg_hbm, fm_hbm, out_hbm, cnt_hbm,
             ids_v, acc_v, cntf_v, cnti_v, scale_v, bufa, bufb,
             sema, semb):
    nc = 2
    wid = lax.axis_index("s") * nc + lax.axis_index("c")
    b = wid

    pltpu.sync_copy(seg_hbm.at[b], ids_v)

    zeros = jnp.zeros((L,), jnp.float32)
    ones = jnp.ones((L,), jnp.float32)

    @plsc.parallel_loop(0, (S * C) // L, unroll=8)
    def _(i):
        acc_v[pl.ds(i * L, L)] = zeros

    for i in range(S // L):
        cntf_v[pl.ds(i * L, L)] = zeros

    @plsc.parallel_loop(0, P // L, unroll=4)
    def _(g):
        ids = ids_v[pl.ds(g * L, L)]
        plsc.addupdate_scatter(cntf_v, [ids], ones)

    def start(chunk, buf, sem):
        pltpu.make_async_copy(
            fm_hbm.at[b, pl.ds(chunk * (CHUNK_C * P), CHUNK_C * P)],
            buf, sem).start()

    start(0, bufa, sema)

    def compute(chunk, buf):
        c0 = chunk * CHUNK_C

        @plsc.parallel_loop(0, P // L, unroll=4)
        def _(g):
            base = ids_v[pl.ds(g * L, L)] + c0 * S
            off = g * L
            for cc in range(CHUNK_C):
                vals = buf[pl.ds(off + cc * P, L)]
                plsc.addupdate_scatter(acc_v, [base + cc * S], vals)

    def mbody(k, _):
        start(2 * k + 1, bufb, semb)
        pltpu.make_async_copy(
            fm_hbm.at[b, pl.ds(0, CHUNK_C * P)], bufa, sema).wait()
        compute(2 * k, bufa)

        @pl.when(k < NCHUNK // 2 - 1)
        def _():
            start(2 * k + 2, bufa, sema)

        pltpu.make_async_copy(
            fm_hbm.at[b, pl.ds(0, CHUNK_C * P)], bufb, semb).wait()
        compute(2 * k + 1, bufb)
        return 0

    lax.fori_loop(0, NCHUNK // 2, mbody, 0)

    for i in range(S // L):
        cnt = cntf_v[pl.ds(i * L, L)]
        sc = jnp.where(cnt >= MINPIX, 1.0 / jnp.maximum(cnt, 1.0), 0.0)
        scale_v[pl.ds(i * L, L)] = sc
        cnti_v[pl.ds(i * L, L)] = cnt.astype(jnp.int32)

    @plsc.parallel_loop(0, C, unroll=2)
    def _(c):
        for j in range(S // L):
            sv = scale_v[pl.ds(j * L, L)]
            o = c * S + j * L
            acc_v[pl.ds(o, L)] = acc_v[pl.ds(o, L)] * sv

    pltpu.sync_copy(acc_v, out_hbm.at[b])
    pltpu.sync_copy(cnti_v, cnt_hbm.at[b])


@jax.jit
def _sc_call(seg, fm):
    mesh = plsc.VectorSubcoreMesh(core_axis_name="c", subcore_axis_name="s")
    f = functools.partial(
        pl.kernel,
        mesh=mesh,
        compiler_params=pltpu.CompilerParams(needs_layout_passes=False),
        out_type=[
            jax.ShapeDtypeStruct((B, C * S), jnp.float32),
            jax.ShapeDtypeStruct((B, S), jnp.int32),
        ],
        scratch_types=[
            pltpu.VMEM((P,), jnp.int32),
            pltpu.VMEM((C * S,), jnp.float32),
            pltpu.VMEM((S,), jnp.float32),
            pltpu.VMEM((S,), jnp.int32),
            pltpu.VMEM((S,), jnp.float32),
            pltpu.VMEM((CHUNK_C * P,), jnp.float32),
            pltpu.VMEM((CHUNK_C * P,), jnp.float32),
            pltpu.SemaphoreType.DMA,
            pltpu.SemaphoreType.DMA,
        ],
    )(_sc_body)
    return f(seg, fm)


def kernel(segment_ids, sam_encoder_embeddings):
    fm = jnp.squeeze(sam_encoder_embeddings, axis=1).reshape(B, C * P)
    seg = segment_ids.reshape(B, P)
    out, cnt = _sc_call(seg, fm)
    valid = cnt >= int(MINPIX)
    return out.reshape(B, C, S).transpose(0, 2, 1), valid


# speedup vs baseline: 2.4295x; 1.0002x over previous
import functools

import jax
import jax.numpy as jnp
from jax import lax
from jax.experimental import pallas as pl
from jax.experimental.pallas import tpu as pltpu
from jax.experimental.pallas import tpu_sc as plsc

B = 32
C = 256
P = 4096
S = 64
L = 16
MINPIX = 16.0
CHUNK_C = 8
NCHUNK = C // CHUNK_C


def _sc_body(seg_hbm, fm_hbm, out_hbm, cnt_hbm,
             ids_v, acc_v, cntf_v, cnti_v, scale_v, bufa, bufb,
             sema, semb):
    nc = 2
    wid = lax.axis_index("s") * nc + lax.axis_index("c")
    b = wid

    pltpu.sync_copy(seg_hbm.at[b], ids_v)

    zeros = jnp.zeros((L,), jnp.float32)
    ones = jnp.ones((L,), jnp.float32)

    @plsc.parallel_loop(0, (S * C) // L, unroll=8)
    def _(i):
        acc_v[pl.ds(i * L, L)] = zeros

    for i in range(S // L):
        cntf_v[pl.ds(i * L, L)] = zeros

    @plsc.parallel_loop(0, P // L, unroll=4)
    def _(g):
        ids = ids_v[pl.ds(g * L, L)]
        plsc.addupdate_scatter(cntf_v, [ids], ones)

    def start(chunk, buf, sem):
        pltpu.make_async_copy(
            fm_hbm.at[b, pl.ds(chunk * (CHUNK_C * P), CHUNK_C * P)],
            buf, sem).start()

    start(0, bufa, sema)

    def compute(chunk, buf):
        c0 = chunk * CHUNK_C

        @plsc.parallel_loop(0, P // L, unroll=8)
        def _(g):
            base = ids_v[pl.ds(g * L, L)] + c0 * S
            off = g * L
            for cc in range(CHUNK_C):
                vals = buf[pl.ds(off + cc * P, L)]
                plsc.addupdate_scatter(acc_v, [base + cc * S], vals)

    def mbody(k, _):
        start(2 * k + 1, bufb, semb)
        pltpu.make_async_copy(
            fm_hbm.at[b, pl.ds(0, CHUNK_C * P)], bufa, sema).wait()
        compute(2 * k, bufa)

        @pl.when(k < NCHUNK // 2 - 1)
        def _():
            start(2 * k + 2, bufa, sema)

        pltpu.make_async_copy(
            fm_hbm.at[b, pl.ds(0, CHUNK_C * P)], bufb, semb).wait()
        compute(2 * k + 1, bufb)
        return 0

    lax.fori_loop(0, NCHUNK // 2, mbody, 0)

    for i in range(S // L):
        cnt = cntf_v[pl.ds(i * L, L)]
        sc = jnp.where(cnt >= MINPIX, 1.0 / jnp.maximum(cnt, 1.0), 0.0)
        scale_v[pl.ds(i * L, L)] = sc
        cnti_v[pl.ds(i * L, L)] = cnt.astype(jnp.int32)

    @plsc.parallel_loop(0, C, unroll=2)
    def _(c):
        for j in range(S // L):
            sv = scale_v[pl.ds(j * L, L)]
            o = c * S + j * L
            acc_v[pl.ds(o, L)] = acc_v[pl.ds(o, L)] * sv

    pltpu.sync_copy(acc_v, out_hbm.at[b])
    pltpu.sync_copy(cnti_v, cnt_hbm.at[b])


@jax.jit
def _sc_call(seg, fm):
    mesh = plsc.VectorSubcoreMesh(core_axis_name="c", subcore_axis_name="s")
    f = functools.partial(
        pl.kernel,
        mesh=mesh,
        compiler_params=pltpu.CompilerParams(needs_layout_passes=False),
        out_type=[
            jax.ShapeDtypeStruct((B, C * S), jnp.float32),
            jax.ShapeDtypeStruct((B, S), jnp.int32),
        ],
        scratch_types=[
            pltpu.VMEM((P,), jnp.int32),
            pltpu.VMEM((C * S,), jnp.float32),
            pltpu.VMEM((S,), jnp.float32),
            pltpu.VMEM((S,), jnp.int32),
            pltpu.VMEM((S,), jnp.float32),
            pltpu.VMEM((CHUNK_C * P,), jnp.float32),
            pltpu.VMEM((CHUNK_C * P,), jnp.float32),
            pltpu.SemaphoreType.DMA,
            pltpu.SemaphoreType.DMA,
        ],
    )(_sc_body)
    return f(seg, fm)


def kernel(segment_ids, sam_encoder_embeddings):
    fm = jnp.squeeze(sam_encoder_embeddings, axis=1).reshape(B, C * P)
    seg = segment_ids.reshape(B, P)
    out, cnt = _sc_call(seg, fm)
    valid = cnt >= int(MINPIX)
    return out.reshape(B, C, S).transpose(0, 2, 1), valid
